# baseline (speedup 1.0000x reference)
import jax
import jax.numpy as jnp
from jax.experimental import pallas as pl
from jax.experimental.pallas import tpu as pltpu

BATCH = 4
SEQ_LEN = 8192
D_MODEL = 768
BLK_S = 1024
BLK_B = 4


def _add_kernel(x_ref, pe_ref, o_ref):
    o_ref[...] = x_ref[...] + pe_ref[...][None]


def kernel(x, pe):
    n_s = SEQ_LEN // BLK_S
    n_b = BATCH // BLK_B
    return pl.pallas_call(
        _add_kernel,
        grid=(n_s, n_b),
        in_specs=[
            pl.BlockSpec((BLK_B, BLK_S, D_MODEL), lambda s, b: (b, s, 0)),
            pl.BlockSpec((BLK_S, D_MODEL), lambda s, b: (s, 0)),
        ],
        out_specs=pl.BlockSpec((BLK_B, BLK_S, D_MODEL), lambda s, b: (b, s, 0)),
        out_shape=jax.ShapeDtypeStruct((BATCH, SEQ_LEN, D_MODEL), x.dtype),
        compiler_params=pltpu.CompilerParams(
            dimension_semantics=("parallel", "parallel")
        ),
    )(x, pe)


# speedup vs baseline: 1.0248x; 1.0248x over previous
import jax
import jax.numpy as jnp
from jax.experimental import pallas as pl
from jax.experimental.pallas import tpu as pltpu

BATCH = 4
SEQ_LEN = 8192
D_MODEL = 768
CH_S = 1024
N_SC = SEQ_LEN // CH_S
NITER = N_SC * BATCH
D_IN = 6
D_OUT = 6


def _body(x_hbm, pe_hbm, o_hbm, xbufs, obufs, pebufs, insems, outsems, pesems):
    def in_copy(k):
        s, b = divmod(k, BATCH)
        slot = k % D_IN
        return pltpu.make_async_copy(
            x_hbm.at[b, pl.ds(s * CH_S, CH_S)], xbufs.at[slot], insems.at[slot]
        )

    def out_copy(k):
        s, b = divmod(k, BATCH)
        slot = k % D_OUT
        return pltpu.make_async_copy(
            obufs.at[slot], o_hbm.at[b, pl.ds(s * CH_S, CH_S)], outsems.at[slot]
        )

    def pe_copy(s):
        return pltpu.make_async_copy(
            pe_hbm.at[pl.ds(s * CH_S, CH_S)], pebufs.at[s % 2], pesems.at[s % 2]
        )

    for k in range(D_IN):
        in_copy(k).start()
    pe_copy(0).start()
    pe_copy(1).start()

    for k in range(NITER):
        s, b = divmod(k, BATCH)
        islot, oslot = k % D_IN, k % D_OUT

        in_copy(k).wait()
        if b == 0:
            pe_copy(s).wait()
        if k >= D_OUT:
            out_copy(k - D_OUT).wait()

        obufs[oslot] = xbufs[islot] + pebufs[s % 2]
        out_copy(k).start()

        if k + D_IN < NITER:
            in_copy(k + D_IN).start()
        if b == BATCH - 1 and s + 2 < N_SC:
            pe_copy(s + 2).start()

    for k in range(max(NITER - D_OUT, 0), NITER):
        out_copy(k).wait()


def kernel(x, pe):
    return pl.pallas_call(
        _body,
        in_specs=[
            pl.BlockSpec(memory_space=pl.ANY),
            pl.BlockSpec(memory_space=pl.ANY),
        ],
        out_specs=pl.BlockSpec(memory_space=pl.ANY),
        out_shape=jax.ShapeDtypeStruct((BATCH, SEQ_LEN, D_MODEL), x.dtype),
        scratch_shapes=[
            pltpu.VMEM((D_IN, CH_S, D_MODEL), jnp.float32),
            pltpu.VMEM((D_OUT, CH_S, D_MODEL), jnp.float32),
            pltpu.VMEM((2, CH_S, D_MODEL), jnp.float32),
            pltpu.SemaphoreType.DMA((D_IN,)),
            pltpu.SemaphoreType.DMA((D_OUT,)),
            pltpu.SemaphoreType.DMA((2,)),
        ],
    )(x, pe)
